# baseline (speedup 1.0000x reference)
import functools

import jax
import jax.numpy as jnp
from jax.experimental import pallas as pl
from jax.experimental.pallas import tpu as pltpu

EPS = 1e-12
CB = 128
BB = 16
NK = 8
NR = 9


def _bn_kernel(v_hbm, p_ref, g_ref, b_ref, o_hbm, slab, ostage,
               in_sems, out_sems, *, n, nc):
    c = pl.program_id(0)

    def region(cidx, k):
        return (NK * cidx + k) % NR

    def in_copy(cidx, k):
        r = region(cidx, k)
        return pltpu.make_async_copy(
            v_hbm.at[pl.ds(k * BB, BB), :, pl.ds(cidx * CB, CB)],
            slab.at[pl.ds(r * BB, BB)],
            in_sems.at[r],
        )

    @pl.when(c == 0)
    def _prologue():
        for k in range(NK):
            in_copy(0, k).start()

    @pl.when(c < nc - 1)
    def _head_start():
        in_copy(c + 1, 0).start()

    p = p_ref[...]
    acc1 = jnp.zeros((1, CB), jnp.float32)
    acc2 = jnp.zeros((1, CB), jnp.float32)
    for k in range(NK):
        in_copy(c, k).wait()
        x = slab[pl.ds(region(c, k) * BB, BB)] + p[None]
        acc1 = acc1 + jnp.sum(x, axis=(0, 1))[None]
        acc2 = acc2 + jnp.sum(x * x, axis=(0, 1))[None]

    mean = acc1 / n
    var = acc2 / n - mean * mean
    sc = g_ref[...] * jax.lax.rsqrt(var + EPS)
    t = p * sc + (b_ref[...] - mean * sc)

    for k in range(NK):
        slot = k % 2

        def _wait_slot(s=slot):
            pltpu.make_async_copy(
                ostage.at[s], o_hbm.at[pl.ds(0, BB), :, pl.ds(0, CB)],
                out_sems.at[s],
            ).wait()

        if k >= 2:
            _wait_slot()
        else:
            pl.when(c > 0)(_wait_slot)

        ostage[slot] = slab[pl.ds(region(c, k) * BB, BB)] * sc[None] + t[None]

        if k < NK - 1:
            @pl.when(c < nc - 1)
            def _prefetch(k=k):
                in_copy(c + 1, k + 1).start()

        pltpu.make_async_copy(
            ostage.at[slot],
            o_hbm.at[pl.ds(k * BB, BB), :, pl.ds(c * CB, CB)],
            out_sems.at[slot],
        ).start()

    @pl.when(c == nc - 1)
    def _drain():
        for s in range(2):
            pltpu.make_async_copy(
                ostage.at[s], o_hbm.at[pl.ds(0, BB), :, pl.ds(0, CB)],
                out_sems.at[s],
            ).wait()


@jax.jit
def _run(batch_video, row_table, col_table, gamma, beta):
    bsz, hsz, height, width = batch_video.shape
    hw = height * width
    n = bsz * hw
    nc = hsz // CB
    v = jnp.transpose(batch_video, (0, 2, 3, 1)).reshape(bsz, hw, hsz)
    r = row_table[:height].reshape(hsz, height).T
    c = col_table[:width].reshape(hsz, width).T
    p = (r[:, None, :] + c[None, :, :]).reshape(hw, hsz)
    g2 = gamma.reshape(1, hsz)
    b2 = beta.reshape(1, hsz)

    out = pl.pallas_call(
        functools.partial(_bn_kernel, n=float(n), nc=nc),
        grid=(nc,),
        in_specs=[
            pl.BlockSpec(memory_space=pl.ANY),
            pl.BlockSpec((hw, CB), lambda i: (0, i)),
            pl.BlockSpec((1, CB), lambda i: (0, i)),
            pl.BlockSpec((1, CB), lambda i: (0, i)),
        ],
        out_specs=pl.BlockSpec(memory_space=pl.ANY),
        out_shape=jax.ShapeDtypeStruct((bsz, hw, hsz), batch_video.dtype),
        scratch_shapes=[
            pltpu.VMEM((NR * BB, hw, CB), jnp.float32),
            pltpu.VMEM((2, BB, hw, CB), jnp.float32),
            pltpu.SemaphoreType.DMA((NR,)),
            pltpu.SemaphoreType.DMA((2,)),
        ],
    )(v, p, g2, b2)
    return jnp.transpose(out.reshape(bsz, height, width, hsz), (0, 3, 1, 2))


def kernel(batch_video, row_table, col_table, gamma, beta):
    return _run(batch_video, row_table, col_table, gamma, beta)


# speedup vs baseline: 1.0354x; 1.0354x over previous
import functools

import jax
import jax.numpy as jnp
from jax.experimental import pallas as pl
from jax.experimental.pallas import tpu as pltpu

EPS = 1e-12
CB = 128
BB = 32
NK = 4


def _bn_kernel(v_hbm, p_ref, g_ref, b_ref, o_hbm, slab, ostage,
               in_sems, out_sems, *, n, nc):
    c = pl.program_id(0)

    def in_copy(cidx, k):
        return pltpu.make_async_copy(
            v_hbm.at[pl.ds(k * BB, BB), :, pl.ds(cidx * CB, CB)],
            slab.at[pl.ds(k * BB, BB)],
            in_sems.at[k],
        )

    @pl.when(c == 0)
    def _prologue():
        for k in range(NK):
            in_copy(0, k).start()

    p = p_ref[...]
    acc1 = jnp.zeros((1, CB), jnp.float32)
    acc2 = jnp.zeros((1, CB), jnp.float32)
    for k in range(NK):
        in_copy(c, k).wait()
        x = slab[pl.ds(k * BB, BB)] + p[None]
        acc1 = acc1 + jnp.sum(x, axis=(0, 1))[None]
        acc2 = acc2 + jnp.sum(x * x, axis=(0, 1))[None]

    mean = acc1 / n
    var = acc2 / n - mean * mean
    sc = g_ref[...] * jax.lax.rsqrt(var + EPS)
    t = p * sc + (b_ref[...] - mean * sc)

    for k in range(NK):
        slot = k % 2

        def _wait_slot(s=slot):
            pltpu.make_async_copy(
                ostage.at[s], o_hbm.at[pl.ds(0, BB), :, pl.ds(0, CB)],
                out_sems.at[s],
            ).wait()

        if k >= 2:
            _wait_slot()
        else:
            pl.when(c > 0)(_wait_slot)

        ostage[slot] = slab[pl.ds(k * BB, BB)] * sc[None] + t[None]

        @pl.when(c < nc - 1)
        def _prefetch(k=k):
            in_copy(c + 1, k).start()

        pltpu.make_async_copy(
            ostage.at[slot],
            o_hbm.at[pl.ds(k * BB, BB), :, pl.ds(c * CB, CB)],
            out_sems.at[slot],
        ).start()

    @pl.when(c == nc - 1)
    def _drain():
        for s in range(2):
            pltpu.make_async_copy(
                ostage.at[s], o_hbm.at[pl.ds(0, BB), :, pl.ds(0, CB)],
                out_sems.at[s],
            ).wait()


@jax.jit
def _run(batch_video, row_table, col_table, gamma, beta):
    bsz, hsz, height, width = batch_video.shape
    hw = height * width
    n = bsz * hw
    nc = hsz // CB
    v = jnp.transpose(batch_video, (0, 2, 3, 1)).reshape(bsz, hw, hsz)
    r = row_table[:height].reshape(hsz, height).T
    c = col_table[:width].reshape(hsz, width).T
    p = (r[:, None, :] + c[None, :, :]).reshape(hw, hsz)
    g2 = gamma.reshape(1, hsz)
    b2 = beta.reshape(1, hsz)

    out = pl.pallas_call(
        functools.partial(_bn_kernel, n=float(n), nc=nc),
        grid=(nc,),
        in_specs=[
            pl.BlockSpec(memory_space=pl.ANY),
            pl.BlockSpec((hw, CB), lambda i: (0, i)),
            pl.BlockSpec((1, CB), lambda i: (0, i)),
            pl.BlockSpec((1, CB), lambda i: (0, i)),
        ],
        out_specs=pl.BlockSpec(memory_space=pl.ANY),
        out_shape=jax.ShapeDtypeStruct((bsz, hw, hsz), batch_video.dtype),
        scratch_shapes=[
            pltpu.VMEM((bsz, hw, CB), jnp.float32),
            pltpu.VMEM((2, BB, hw, CB), jnp.float32),
            pltpu.SemaphoreType.DMA((NK,)),
            pltpu.SemaphoreType.DMA((2,)),
        ],
    )(v, p, g2, b2)
    return jnp.transpose(out.reshape(bsz, height, width, hsz), (0, 3, 1, 2))


def kernel(batch_video, row_table, col_table, gamma, beta):
    return _run(batch_video, row_table, col_table, gamma, beta)


# speedup vs baseline: 1.1041x; 1.0664x over previous
import functools

import jax
import jax.numpy as jnp
from jax.experimental import pallas as pl
from jax.experimental.pallas import tpu as pltpu

EPS = 1e-12
CB = 128
BB = 16
NK = 8
NS = 4


def _bn_kernel(v_hbm, p_ref, g_ref, b_ref, o_hbm, slab, ostage,
               in_sems, out_sems, *, n, nc):
    c = pl.program_id(0)

    def in_copy(cidx, k):
        return pltpu.make_async_copy(
            v_hbm.at[pl.ds(k * BB, BB), :, pl.ds(cidx * CB, CB)],
            slab.at[pl.ds(k * BB, BB)],
            in_sems.at[k],
        )

    @pl.when(c == 0)
    def _prologue():
        for k in range(NK):
            in_copy(0, k).start()

    p = p_ref[...]
    acc1 = jnp.zeros((1, CB), jnp.float32)
    acc2 = jnp.zeros((1, CB), jnp.float32)
    for k in range(NK):
        in_copy(c, k).wait()
        x = slab[pl.ds(k * BB, BB)] + p[None]
        acc1 = acc1 + jnp.sum(x, axis=(0, 1))[None]
        acc2 = acc2 + jnp.sum(x * x, axis=(0, 1))[None]

    mean = acc1 / n
    var = acc2 / n - mean * mean
    sc = g_ref[...] * jax.lax.rsqrt(var + EPS)
    t = p * sc + (b_ref[...] - mean * sc)

    for k in range(NK):
        slot = k % NS

        def _wait_slot(s=slot):
            pltpu.make_async_copy(
                ostage.at[s], o_hbm.at[pl.ds(0, BB), :, pl.ds(0, CB)],
                out_sems.at[s],
            ).wait()

        if k >= NS:
            _wait_slot()
        else:
            pl.when(c > 0)(_wait_slot)

        ostage[slot] = slab[pl.ds(k * BB, BB)] * sc[None] + t[None]

        @pl.when(c < nc - 1)
        def _prefetch(k=k):
            in_copy(c + 1, k).start()

        pltpu.make_async_copy(
            ostage.at[slot],
            o_hbm.at[pl.ds(k * BB, BB), :, pl.ds(c * CB, CB)],
            out_sems.at[slot],
        ).start()

    @pl.when(c == nc - 1)
    def _drain():
        for s in range(NS):
            pltpu.make_async_copy(
                ostage.at[s], o_hbm.at[pl.ds(0, BB), :, pl.ds(0, CB)],
                out_sems.at[s],
            ).wait()


@jax.jit
def _run(batch_video, row_table, col_table, gamma, beta):
    bsz, hsz, height, width = batch_video.shape
    hw = height * width
    n = bsz * hw
    nc = hsz // CB
    v = jnp.transpose(batch_video, (0, 2, 3, 1)).reshape(bsz, hw, hsz)
    r = row_table[:height].reshape(hsz, height).T
    c = col_table[:width].reshape(hsz, width).T
    p = (r[:, None, :] + c[None, :, :]).reshape(hw, hsz)
    g2 = gamma.reshape(1, hsz)
    b2 = beta.reshape(1, hsz)

    out = pl.pallas_call(
        functools.partial(_bn_kernel, n=float(n), nc=nc),
        grid=(nc,),
        in_specs=[
            pl.BlockSpec(memory_space=pl.ANY),
            pl.BlockSpec((hw, CB), lambda i: (0, i)),
            pl.BlockSpec((1, CB), lambda i: (0, i)),
            pl.BlockSpec((1, CB), lambda i: (0, i)),
        ],
        out_specs=pl.BlockSpec(memory_space=pl.ANY),
        out_shape=jax.ShapeDtypeStruct((bsz, hw, hsz), batch_video.dtype),
        scratch_shapes=[
            pltpu.VMEM((bsz, hw, CB), jnp.float32),
            pltpu.VMEM((4, BB, hw, CB), jnp.float32),
            pltpu.SemaphoreType.DMA((NK,)),
            pltpu.SemaphoreType.DMA((4,)),
        ],
    )(v, p, g2, b2)
    return jnp.transpose(out.reshape(bsz, height, width, hsz), (0, 3, 1, 2))


def kernel(batch_video, row_table, col_table, gamma, beta):
    return _run(batch_video, row_table, col_table, gamma, beta)
